# Initial kernel scaffold; baseline (speedup 1.0000x reference)
import jax
import jax.numpy as jnp
from jax.experimental import pallas as pl


def kernel(x, centers):
    raise NotImplementedError("write your pallas kernel here")


# speedup vs baseline: 7.6383x; 7.6383x over previous
import jax
import jax.numpy as jnp
from jax.experimental import pallas as pl

N_POINTS = 4096
N_CLUSTERS = 512
N_INPUT = 64
BLOCK_ROWS = 256

_DIMS = (((1,), (1,)), ((), ()))


def _kmeans_assign_kernel(x_ref, c_ref, out_ref):
    x = x_ref[...]
    c = c_ref[...]
    scores = jax.lax.dot_general(
        x, c, dimension_numbers=_DIMS, preferred_element_type=jnp.float32,
        precision=jax.lax.Precision.HIGHEST,
    )
    ones = jnp.ones((8, N_INPUT), jnp.float32)
    c_norm = jax.lax.dot_general(
        ones, c * c, dimension_numbers=_DIMS,
        preferred_element_type=jnp.float32,
        precision=jax.lax.Precision.HIGHEST,
    )
    dist = c_norm[0:1, :] - 2.0 * scores
    m = jnp.min(dist, axis=1, keepdims=True)
    idx = jax.lax.broadcasted_iota(jnp.int32, dist.shape, 1)
    cand = jnp.where(dist == m, idx, N_CLUSTERS)
    out_ref[...] = jnp.min(cand, axis=1, keepdims=True)


def kernel(x, centers):
    out = pl.pallas_call(
        _kmeans_assign_kernel,
        grid=(N_POINTS // BLOCK_ROWS,),
        in_specs=[
            pl.BlockSpec((BLOCK_ROWS, N_INPUT), lambda i: (i, 0)),
            pl.BlockSpec((N_CLUSTERS, N_INPUT), lambda i: (0, 0)),
        ],
        out_specs=pl.BlockSpec((BLOCK_ROWS, 1), lambda i: (i, 0)),
        out_shape=jax.ShapeDtypeStruct((N_POINTS, 1), jnp.int32),
    )(x, centers)
    return out.reshape(N_POINTS)


# speedup vs baseline: 9.1119x; 1.1929x over previous
import jax
import jax.numpy as jnp
from jax.experimental import pallas as pl

N_POINTS = 4096
N_CLUSTERS = 512
N_INPUT = 64
BLOCK_ROWS = 256

_DIMS = (((1,), (1,)), ((), ()))


def _hi_lo(v):
    hi = v.astype(jnp.bfloat16)
    lo = (v - hi.astype(jnp.float32)).astype(jnp.bfloat16)
    return hi, lo


def _kmeans_assign_kernel(xp_ref, cp_ref, out_ref):
    xp = xp_ref[...]
    cp = cp_ref[...]
    dist = jax.lax.dot_general(
        xp, cp, dimension_numbers=_DIMS, preferred_element_type=jnp.float32,
    )
    m = jnp.min(dist, axis=1, keepdims=True)
    idx = jax.lax.broadcasted_iota(jnp.int32, dist.shape, 1)
    cand = jnp.where(dist == m, idx, N_CLUSTERS)
    out_ref[...] = jnp.min(cand, axis=1, keepdims=True)


def kernel(x, centers):
    x = x.astype(jnp.float32)
    centers = centers.astype(jnp.float32)
    xh, xl = _hi_lo(-2.0 * x)
    ch, cl = _hi_lo(centers)
    cn = jnp.sum(centers * centers, axis=1, keepdims=True)
    cnh, cnl = _hi_lo(cn)
    ones = jnp.ones((N_POINTS, 1), jnp.bfloat16)
    x_pack = jnp.concatenate([xh, xl, xh, ones, ones], axis=1)
    c_pack = jnp.concatenate([ch, ch, cl, cnh, cnl], axis=1)

    out = pl.pallas_call(
        _kmeans_assign_kernel,
        grid=(N_POINTS // BLOCK_ROWS,),
        in_specs=[
            pl.BlockSpec((BLOCK_ROWS, 194), lambda i: (i, 0)),
            pl.BlockSpec((N_CLUSTERS, 194), lambda i: (0, 0)),
        ],
        out_specs=pl.BlockSpec((BLOCK_ROWS, 1), lambda i: (i, 0)),
        out_shape=jax.ShapeDtypeStruct((N_POINTS, 1), jnp.int32),
    )(x_pack, c_pack)
    return out.reshape(N_POINTS)


# speedup vs baseline: 10.2195x; 1.1216x over previous
import jax
import jax.numpy as jnp
from jax.experimental import pallas as pl
from jax.experimental.pallas import tpu as pltpu

N_POINTS = 4096
N_CLUSTERS = 512
N_INPUT = 64
BLOCK_ROWS = 512

_DIMS = (((1,), (1,)), ((), ()))


def _kmeans_assign_kernel(x_ref, c_ref, out_ref, cn_ref):
    @pl.when(pl.program_id(0) == 0)
    def _():
        c = c_ref[...]
        ones = jnp.ones((8, N_INPUT), jnp.float32)
        cn_ref[...] = jax.lax.dot_general(
            ones, c * c, dimension_numbers=_DIMS,
            preferred_element_type=jnp.float32,
            precision=jax.lax.Precision.HIGHEST,
        )

    x = x_ref[...]
    c = c_ref[...]
    scores = jax.lax.dot_general(
        x, c, dimension_numbers=_DIMS, preferred_element_type=jnp.float32,
        precision=jax.lax.Precision.HIGHEST,
    )
    dist = cn_ref[0:1, :] - 2.0 * scores
    m = jnp.min(dist, axis=1, keepdims=True)
    idx = jax.lax.broadcasted_iota(jnp.int32, dist.shape, 1)
    cand = jnp.where(dist == m, idx, N_CLUSTERS)
    out_ref[...] = jnp.min(cand, axis=1, keepdims=True)


def kernel(x, centers):
    out = pl.pallas_call(
        _kmeans_assign_kernel,
        grid=(N_POINTS // BLOCK_ROWS,),
        in_specs=[
            pl.BlockSpec((BLOCK_ROWS, N_INPUT), lambda i: (i, 0)),
            pl.BlockSpec((N_CLUSTERS, N_INPUT), lambda i: (0, 0)),
        ],
        out_specs=pl.BlockSpec((BLOCK_ROWS, 1), lambda i: (i, 0)),
        out_shape=jax.ShapeDtypeStruct((N_POINTS, 1), jnp.int32),
        scratch_shapes=[pltpu.VMEM((8, N_CLUSTERS), jnp.float32)],
    )(x, centers)
    return out.reshape(N_POINTS)


# speedup vs baseline: 10.5458x; 1.0319x over previous
import jax
import jax.numpy as jnp
from jax.experimental import pallas as pl
from jax.experimental.pallas import tpu as pltpu

N_POINTS = 4096
N_CLUSTERS = 512
N_INPUT = 64
BLOCK_ROWS = 4096

_DIMS = (((1,), (1,)), ((), ()))


def _kmeans_assign_kernel(x_ref, c_ref, out_ref, cn_ref):
    @pl.when(pl.program_id(0) == 0)
    def _():
        c = c_ref[...]
        ones = jnp.ones((8, N_INPUT), jnp.float32)
        cn_ref[...] = jax.lax.dot_general(
            ones, c * c, dimension_numbers=_DIMS,
            preferred_element_type=jnp.float32,
            precision=jax.lax.Precision.HIGHEST,
        )

    x = x_ref[...]
    c = c_ref[...]
    scores = jax.lax.dot_general(
        x, c, dimension_numbers=_DIMS, preferred_element_type=jnp.float32,
        precision=jax.lax.Precision.HIGHEST,
    )
    dist = cn_ref[0:1, :] - 2.0 * scores
    m = jnp.min(dist, axis=1, keepdims=True)
    idx = jax.lax.broadcasted_iota(jnp.int32, dist.shape, 1)
    cand = jnp.where(dist == m, idx, N_CLUSTERS)
    out_ref[...] = jnp.min(cand, axis=1, keepdims=True)


def kernel(x, centers):
    out = pl.pallas_call(
        _kmeans_assign_kernel,
        grid=(N_POINTS // BLOCK_ROWS,),
        in_specs=[
            pl.BlockSpec((BLOCK_ROWS, N_INPUT), lambda i: (i, 0)),
            pl.BlockSpec((N_CLUSTERS, N_INPUT), lambda i: (0, 0)),
        ],
        out_specs=pl.BlockSpec((BLOCK_ROWS, 1), lambda i: (i, 0)),
        out_shape=jax.ShapeDtypeStruct((N_POINTS, 1), jnp.int32),
        scratch_shapes=[pltpu.VMEM((8, N_CLUSTERS), jnp.float32)],
    )(x, centers)
    return out.reshape(N_POINTS)


# speedup vs baseline: 14.0838x; 1.3355x over previous
import jax
import jax.numpy as jnp
from jax.experimental import pallas as pl

N_POINTS = 4096
N_CLUSTERS = 512
N_INPUT = 64
BLOCK_COLS = 4096

_DIMS = (((1,), (1,)), ((), ()))


def _kmeans_assign_kernel(c_ref, x_ref, out_ref):
    c = c_ref[...]
    x = x_ref[...]
    scores = jax.lax.dot_general(
        c, x, dimension_numbers=_DIMS, preferred_element_type=jnp.float32,
        precision=jax.lax.Precision.HIGHEST,
    )
    cn = jnp.sum(c * c, axis=1, keepdims=True)
    dist = cn - 2.0 * scores
    m = jnp.min(dist, axis=0, keepdims=True)
    idx = jax.lax.broadcasted_iota(jnp.int32, dist.shape, 0)
    cand = jnp.where(dist == m, idx, N_CLUSTERS)
    out_ref[...] = jnp.min(cand, axis=0, keepdims=True)


def kernel(x, centers):
    out = pl.pallas_call(
        _kmeans_assign_kernel,
        grid=(N_POINTS // BLOCK_COLS,),
        in_specs=[
            pl.BlockSpec((N_CLUSTERS, N_INPUT), lambda i: (0, 0)),
            pl.BlockSpec((BLOCK_COLS, N_INPUT), lambda i: (i, 0)),
        ],
        out_specs=pl.BlockSpec((1, BLOCK_COLS), lambda i: (0, i)),
        out_shape=jax.ShapeDtypeStruct((1, N_POINTS), jnp.int32),
    )(centers, x)
    return out.reshape(N_POINTS)
